# Initial kernel scaffold; baseline (speedup 1.0000x reference)
import jax
import jax.numpy as jnp
from jax.experimental import pallas as pl


def kernel(h, edge_index):
    raise NotImplementedError("write your pallas kernel here")


# speedup vs baseline: 1.1816x; 1.1816x over previous
import functools

import jax
import jax.numpy as jnp
from jax import lax
from jax.experimental import pallas as pl
from jax.experimental.pallas import tpu as pltpu
from jax.experimental.pallas import tpu_sc as plsc

_NC = 2
_NS = 16
_NW = _NC * _NS
_L = 16


def kernel(h, edge_index):
    n_nodes, d_feat = h.shape
    n_edges = edge_index.shape[1]
    src = edge_index[0].astype(jnp.int32)
    dst = edge_index[1].astype(jnp.int32)

    e_per = n_edges // _NW
    chunk = 80
    n_chunks = e_per // chunk

    mesh = plsc.VectorSubcoreMesh(core_axis_name="c", subcore_axis_name="s")

    @functools.partial(
        pl.kernel,
        out_type=jax.ShapeDtypeStruct((n_edges,), jnp.float32),
        mesh=mesh,
        scratch_types=[
            pltpu.VMEM((e_per,), jnp.int32),
            pltpu.VMEM((e_per,), jnp.int32),
            pltpu.VMEM((chunk, d_feat), jnp.float32),
            pltpu.VMEM((chunk, d_feat), jnp.float32),
            pltpu.VMEM((chunk,), jnp.float32),
            pltpu.SemaphoreType.DMA,
            pltpu.SemaphoreType.DMA,
        ],
        compiler_params=pltpu.CompilerParams(needs_layout_passes=False),
    )
    def sc_kernel(h_hbm, src_hbm, dst_hbm, out_hbm,
                  src_v, dst_v, srows, drows, sco, sem_s, sem_d):
        wid = lax.axis_index("s") * _NC + lax.axis_index("c")
        base = wid * e_per
        pltpu.sync_copy(src_hbm.at[pl.ds(base, e_per)], src_v)
        pltpu.sync_copy(dst_hbm.at[pl.ds(base, e_per)], dst_v)

        lane = lax.iota(jnp.int32, _L)

        def chunk_body(c, carry):
            cs = c * chunk
            cp_s = pltpu.async_copy(
                h_hbm.at[src_v.at[pl.ds(cs, chunk)]], srows, sem_s)
            cp_d = pltpu.async_copy(
                h_hbm.at[dst_v.at[pl.ds(cs, chunk)]], drows, sem_d)
            cp_s.wait()
            cp_d.wait()

            def group_body(g, carry2):
                rows = lane + g * _L
                acc = jnp.zeros((_L,), jnp.float32)
                for dcol in range(d_feat):
                    cols = jnp.full((_L,), dcol, jnp.int32)
                    a = plsc.load_gather(srows, [rows, cols])
                    b = plsc.load_gather(drows, [rows, cols])
                    acc = acc + a * b
                sco[pl.ds(g * _L, _L)] = acc
                return carry2

            lax.fori_loop(0, chunk // _L, group_body, 0)
            pltpu.sync_copy(sco, out_hbm.at[pl.ds(base + cs, chunk)])
            return carry

        lax.fori_loop(0, n_chunks, chunk_body, 0)

    return sc_kernel(h, src, dst)


# speedup vs baseline: 5.4319x; 4.5971x over previous
import functools

import jax
import jax.numpy as jnp
from jax import lax
from jax.experimental import pallas as pl
from jax.experimental.pallas import tpu as pltpu
from jax.experimental.pallas import tpu_sc as plsc

_NC = 2
_NS = 16
_NW = _NC * _NS
_L = 16


def kernel(h, edge_index):
    n_nodes, d_feat = h.shape
    n_edges = edge_index.shape[1]
    src = edge_index[0].astype(jnp.int32)
    dst = edge_index[1].astype(jnp.int32)

    e_per = n_edges // _NW
    chunk = 80
    n_chunks = e_per // chunk

    mesh = plsc.VectorSubcoreMesh(core_axis_name="c", subcore_axis_name="s")

    @functools.partial(
        pl.kernel,
        out_type=jax.ShapeDtypeStruct((n_edges,), jnp.float32),
        mesh=mesh,
        scratch_types=[
            pltpu.VMEM((e_per,), jnp.int32),
            pltpu.VMEM((e_per,), jnp.int32),
            pltpu.VMEM((chunk, d_feat), jnp.float32),
            pltpu.VMEM((chunk, d_feat), jnp.float32),
            pltpu.VMEM((chunk,), jnp.float32),
            pltpu.SemaphoreType.DMA,
            pltpu.SemaphoreType.DMA,
        ],
        compiler_params=pltpu.CompilerParams(needs_layout_passes=False),
    )
    def sc_kernel(h_hbm, src_hbm, dst_hbm, out_hbm,
                  src_v, dst_v, srows, drows, sco, sem_s, sem_d):
        wid = lax.axis_index("s") * _NC + lax.axis_index("c")
        base = wid * e_per
        pltpu.sync_copy(src_hbm.at[pl.ds(base, e_per)], src_v)
        pltpu.sync_copy(dst_hbm.at[pl.ds(base, e_per)], dst_v)

        lane = lax.iota(jnp.int32, _L)
        lane_xor = [lane ^ j for j in range(8)]

        def chunk_body(c, carry):
            cs = c * chunk
            cp_s = pltpu.async_copy(
                h_hbm.at[src_v.at[pl.ds(cs, chunk)]], srows, sem_s)
            cp_d = pltpu.async_copy(
                h_hbm.at[dst_v.at[pl.ds(cs, chunk)]], drows, sem_d)
            cp_s.wait()
            cp_d.wait()

            def group_body(g, carry2):
                rows = lane + g * _L
                def octave(o, accs):
                    dbase = o * 8
                    a0, a1, a2, a3 = accs
                    for j in range(8):
                        cols = lane_xor[j] ^ dbase
                        a = plsc.load_gather(srows, [rows, cols])
                        b = plsc.load_gather(drows, [rows, cols])
                        if j % 4 == 0:
                            a0 = a0 + a * b
                        elif j % 4 == 1:
                            a1 = a1 + a * b
                        elif j % 4 == 2:
                            a2 = a2 + a * b
                        else:
                            a3 = a3 + a * b
                    return a0, a1, a2, a3

                zero = jnp.zeros((_L,), jnp.float32)
                a0, a1, a2, a3 = lax.fori_loop(
                    0, d_feat // 8, octave, (zero, zero, zero, zero))
                sco[pl.ds(g * _L, _L)] = (a0 + a1) + (a2 + a3)
                return carry2

            lax.fori_loop(0, chunk // _L, group_body, 0)
            pltpu.sync_copy(sco, out_hbm.at[pl.ds(base + cs, chunk)])
            return carry

        lax.fori_loop(0, n_chunks, chunk_body, 0)

    return sc_kernel(h, src, dst)


# speedup vs baseline: 9.2870x; 1.7097x over previous
import functools

import jax
import jax.numpy as jnp
from jax import lax
from jax.experimental import pallas as pl
from jax.experimental.pallas import tpu as pltpu
from jax.experimental.pallas import tpu_sc as plsc

_NC = 2
_NS = 16
_NW = _NC * _NS
_L = 16


def kernel(h, edge_index):
    n_nodes, d_feat = h.shape
    n_edges = edge_index.shape[1]
    src = edge_index[0].astype(jnp.int32)
    dst = edge_index[1].astype(jnp.int32)

    e_per = n_edges // _NW
    chunk = 80
    n_chunks = e_per // chunk

    mesh = plsc.VectorSubcoreMesh(core_axis_name="c", subcore_axis_name="s")

    @functools.partial(
        pl.kernel,
        out_type=jax.ShapeDtypeStruct((n_edges,), jnp.float32),
        mesh=mesh,
        scratch_types=[
            pltpu.VMEM((e_per,), jnp.int32),
            pltpu.VMEM((e_per,), jnp.int32),
            pltpu.VMEM((chunk, d_feat), jnp.float32),
            pltpu.VMEM((chunk, d_feat), jnp.float32),
            pltpu.VMEM((chunk, d_feat), jnp.float32),
            pltpu.VMEM((chunk, d_feat), jnp.float32),
            pltpu.VMEM((e_per,), jnp.float32),
            pltpu.SemaphoreType.DMA,
            pltpu.SemaphoreType.DMA,
            pltpu.SemaphoreType.DMA,
            pltpu.SemaphoreType.DMA,
        ],
        compiler_params=pltpu.CompilerParams(needs_layout_passes=False),
    )
    def sc_kernel(h_hbm, src_hbm, dst_hbm, out_hbm,
                  src_v, dst_v, sbuf0, dbuf0, sbuf1, dbuf1, sco,
                  sem_s0, sem_d0, sem_s1, sem_d1):
        wid = lax.axis_index("s") * _NC + lax.axis_index("c")
        base = wid * e_per
        pltpu.sync_copy(src_hbm.at[pl.ds(base, e_per)], src_v)
        pltpu.sync_copy(dst_hbm.at[pl.ds(base, e_per)], dst_v)

        lane = lax.iota(jnp.int32, _L)
        lane_xor = [lane ^ j for j in range(8)]

        def start_gather(c, sbuf, dbuf, sem_s, sem_d):
            cs = c * chunk
            pltpu.async_copy(h_hbm.at[src_v.at[pl.ds(cs, chunk)]], sbuf, sem_s)
            pltpu.async_copy(h_hbm.at[dst_v.at[pl.ds(cs, chunk)]], dbuf, sem_d)

        def wait_gather(sbuf, dbuf, sem_s, sem_d):
            pltpu.make_async_copy(h_hbm.at[src_v.at[pl.ds(0, chunk)]],
                                  sbuf, sem_s).wait()
            pltpu.make_async_copy(h_hbm.at[dst_v.at[pl.ds(0, chunk)]],
                                  dbuf, sem_d).wait()

        def compute(c, srows, drows):
            cs = c * chunk

            def group_body(g, carry2):
                rows = lane + g * _L

                def octave(o, accs):
                    dbase = o * 8
                    a0, a1, a2, a3 = accs
                    for j in range(8):
                        cols = lane_xor[j] ^ dbase
                        a = plsc.load_gather(srows, [rows, cols])
                        b = plsc.load_gather(drows, [rows, cols])
                        if j % 4 == 0:
                            a0 = a0 + a * b
                        elif j % 4 == 1:
                            a1 = a1 + a * b
                        elif j % 4 == 2:
                            a2 = a2 + a * b
                        else:
                            a3 = a3 + a * b
                    return a0, a1, a2, a3

                zero = jnp.zeros((_L,), jnp.float32)
                a0, a1, a2, a3 = lax.fori_loop(
                    0, d_feat // 8, octave, (zero, zero, zero, zero))
                sco[pl.ds(cs + g * _L, _L)] = (a0 + a1) + (a2 + a3)
                return carry2

            lax.fori_loop(0, chunk // _L, group_body, 0)

        start_gather(0, sbuf0, dbuf0, sem_s0, sem_d0)

        def pair_body(c2, carry):
            a = 2 * c2
            start_gather(a + 1, sbuf1, dbuf1, sem_s1, sem_d1)
            wait_gather(sbuf0, dbuf0, sem_s0, sem_d0)
            compute(a, sbuf0, dbuf0)
            start_gather(a + 2, sbuf0, dbuf0, sem_s0, sem_d0)
            wait_gather(sbuf1, dbuf1, sem_s1, sem_d1)
            compute(a + 1, sbuf1, dbuf1)
            return carry

        lax.fori_loop(0, n_chunks // 2, pair_body, 0)

        wait_gather(sbuf0, dbuf0, sem_s0, sem_d0)
        compute(n_chunks - 1, sbuf0, dbuf0)

        pltpu.sync_copy(sco, out_hbm.at[pl.ds(base, e_per)])

    return sc_kernel(h, src, dst)


# speedup vs baseline: 10.6866x; 1.1507x over previous
import functools

import jax
import jax.numpy as jnp
from jax import lax
from jax.experimental import pallas as pl
from jax.experimental.pallas import tpu as pltpu
from jax.experimental.pallas import tpu_sc as plsc

_NC = 2
_NS = 16
_NW = _NC * _NS
_L = 16


def kernel(h, edge_index):
    n_nodes, d_feat = h.shape
    n_edges = edge_index.shape[1]
    src = edge_index[0].astype(jnp.int32)
    dst = edge_index[1].astype(jnp.int32)
    d_words = d_feat // 2
    h_packed = lax.bitcast_convert_type(
        h.astype(jnp.bfloat16).reshape(n_nodes, d_words, 2), jnp.int32)

    e_per = n_edges // _NW
    chunk = 80
    n_chunks = e_per // chunk

    mesh = plsc.VectorSubcoreMesh(core_axis_name="c", subcore_axis_name="s")

    @functools.partial(
        pl.kernel,
        out_type=jax.ShapeDtypeStruct((n_edges,), jnp.float32),
        mesh=mesh,
        scratch_types=[
            pltpu.VMEM((e_per,), jnp.int32),
            pltpu.VMEM((e_per,), jnp.int32),
            pltpu.VMEM((chunk, d_words), jnp.int32),
            pltpu.VMEM((chunk, d_words), jnp.int32),
            pltpu.VMEM((chunk, d_words), jnp.int32),
            pltpu.VMEM((chunk, d_words), jnp.int32),
            pltpu.VMEM((e_per,), jnp.float32),
            pltpu.SemaphoreType.DMA,
            pltpu.SemaphoreType.DMA,
            pltpu.SemaphoreType.DMA,
            pltpu.SemaphoreType.DMA,
        ],
        compiler_params=pltpu.CompilerParams(
            needs_layout_passes=False, use_tc_tiling_on_sc=False),
    )
    def sc_kernel(h_hbm, src_hbm, dst_hbm, out_hbm,
                  src_v, dst_v, sbuf0, dbuf0, sbuf1, dbuf1, sco,
                  sem_s0, sem_d0, sem_s1, sem_d1):
        wid = lax.axis_index("s") * _NC + lax.axis_index("c")
        base = wid * e_per
        pltpu.sync_copy(src_hbm.at[pl.ds(base, e_per)], src_v)
        pltpu.sync_copy(dst_hbm.at[pl.ds(base, e_per)], dst_v)

        lane = lax.iota(jnp.int32, _L)
        lane_xor = [lane ^ j for j in range(8)]

        def start_gather(c, sbuf, dbuf, sem_s, sem_d):
            cs = c * chunk
            pltpu.async_copy(h_hbm.at[src_v.at[pl.ds(cs, chunk)]], sbuf, sem_s)
            pltpu.async_copy(h_hbm.at[dst_v.at[pl.ds(cs, chunk)]], dbuf, sem_d)

        def wait_gather(sbuf, dbuf, sem_s, sem_d):
            pltpu.make_async_copy(h_hbm.at[src_v.at[pl.ds(0, chunk)]],
                                  sbuf, sem_s).wait()
            pltpu.make_async_copy(h_hbm.at[dst_v.at[pl.ds(0, chunk)]],
                                  dbuf, sem_d).wait()

        def compute(c, srows, drows):
            cs = c * chunk

            def group_body(g, carry2):
                rows = lane + g * _L

                def octave(o, accs):
                    wbase = o * 8
                    acc0, acc1 = accs
                    accbf = None
                    for j in range(8):
                        cols = lane_xor[j] ^ wbase
                        a = plsc.load_gather(srows, [rows, cols])
                        b = plsc.load_gather(drows, [rows, cols])
                        p = (plsc.bitcast(a, jnp.bfloat16)
                             * plsc.bitcast(b, jnp.bfloat16))
                        accbf = p if accbf is None else accbf + p
                    lo, hi = plsc.unpack(accbf, format=plsc.PackFormat.INTERLEAVED)
                    return acc0 + lo, acc1 + hi

                zero = jnp.zeros((_L,), jnp.float32)
                acc0, acc1 = lax.fori_loop(
                    0, d_words // 8, octave, (zero, zero))
                sco[pl.ds(cs + g * _L, _L)] = acc0 + acc1
                return carry2

            lax.fori_loop(0, chunk // _L, group_body, 0)

        start_gather(0, sbuf0, dbuf0, sem_s0, sem_d0)

        def pair_body(c2, carry):
            a = 2 * c2
            start_gather(a + 1, sbuf1, dbuf1, sem_s1, sem_d1)
            wait_gather(sbuf0, dbuf0, sem_s0, sem_d0)
            compute(a, sbuf0, dbuf0)
            start_gather(a + 2, sbuf0, dbuf0, sem_s0, sem_d0)
            wait_gather(sbuf1, dbuf1, sem_s1, sem_d1)
            compute(a + 1, sbuf1, dbuf1)
            return carry

        lax.fori_loop(0, n_chunks // 2, pair_body, 0)

        wait_gather(sbuf0, dbuf0, sem_s0, sem_d0)
        compute(n_chunks - 1, sbuf0, dbuf0)

        pltpu.sync_copy(sco, out_hbm.at[pl.ds(base, e_per)])

    return sc_kernel(h_packed, src, dst)


# speedup vs baseline: 11.4612x; 1.0725x over previous
import functools

import jax
import jax.numpy as jnp
from jax import lax
from jax.experimental import pallas as pl
from jax.experimental.pallas import tpu as pltpu
from jax.experimental.pallas import tpu_sc as plsc

_NC = 2
_NS = 16
_NW = _NC * _NS
_L = 16


def kernel(h, edge_index):
    n_nodes, d_feat = h.shape
    n_edges = edge_index.shape[1]
    src = edge_index[0].astype(jnp.int32)
    dst = edge_index[1].astype(jnp.int32)
    d_words = d_feat // 2
    h_packed = lax.bitcast_convert_type(
        h.astype(jnp.bfloat16).reshape(n_nodes, d_words, 2), jnp.int32)

    e_per = n_edges // _NW
    chunk = 224
    n_full = e_per // chunk
    tail = e_per - n_full * chunk

    mesh = plsc.VectorSubcoreMesh(core_axis_name="c", subcore_axis_name="s")

    @functools.partial(
        pl.kernel,
        out_type=jax.ShapeDtypeStruct((n_edges,), jnp.float32),
        mesh=mesh,
        scratch_types=[
            pltpu.VMEM((e_per,), jnp.int32),
            pltpu.VMEM((e_per,), jnp.int32),
            pltpu.VMEM((chunk, d_words), jnp.int32),
            pltpu.VMEM((chunk, d_words), jnp.int32),
            pltpu.VMEM((chunk, d_words), jnp.int32),
            pltpu.VMEM((chunk, d_words), jnp.int32),
            pltpu.VMEM((e_per,), jnp.float32),
            pltpu.VMEM_SHARED((n_nodes, d_words), jnp.int32),
            pltpu.SemaphoreType.DMA,
            pltpu.SemaphoreType.DMA,
            pltpu.SemaphoreType.DMA,
            pltpu.SemaphoreType.DMA,
        ],
        compiler_params=pltpu.CompilerParams(
            needs_layout_passes=False, use_tc_tiling_on_sc=False),
    )
    def sc_kernel(h_hbm, src_hbm, dst_hbm, out_hbm,
                  src_v, dst_v, sbuf0, dbuf0, sbuf1, dbuf1, sco, h_spmem,
                  sem_s0, sem_d0, sem_s1, sem_d1):
        wid = lax.axis_index("s") * _NC + lax.axis_index("c")
        base = wid * e_per
        pltpu.sync_copy(src_hbm.at[pl.ds(base, e_per)], src_v)
        pltpu.sync_copy(dst_hbm.at[pl.ds(base, e_per)], dst_v)

        @pl.when(lax.axis_index("s") == 0)
        def _stage():
            pltpu.sync_copy(h_hbm, h_spmem)

        plsc.subcore_barrier()

        lane = lax.iota(jnp.int32, _L)
        lane_xor = [lane ^ j for j in range(8)]

        def start_gather(c, sbuf, dbuf, sem_s, sem_d, size):
            cs = c * chunk
            pltpu.async_copy(h_spmem.at[src_v.at[pl.ds(cs, size)]],
                             sbuf.at[pl.ds(0, size)], sem_s)
            pltpu.async_copy(h_spmem.at[dst_v.at[pl.ds(cs, size)]],
                             dbuf.at[pl.ds(0, size)], sem_d)

        def wait_gather(sbuf, dbuf, sem_s, sem_d, size):
            pltpu.make_async_copy(h_spmem.at[src_v.at[pl.ds(0, size)]],
                                  sbuf.at[pl.ds(0, size)], sem_s).wait()
            pltpu.make_async_copy(h_spmem.at[dst_v.at[pl.ds(0, size)]],
                                  dbuf.at[pl.ds(0, size)], sem_d).wait()

        def compute(cs, srows, drows, n_groups):
            def group_body(g, carry2):
                rows = lane + g * _L

                def quad(o, accs):
                    acc0, acc1 = accs
                    for half in range(2):
                        wbase = o * 16 + half * 8
                        accbf = None
                        for j in range(8):
                            cols = lane_xor[j] ^ wbase
                            a = plsc.load_gather(srows, [rows, cols])
                            b = plsc.load_gather(drows, [rows, cols])
                            p = (plsc.bitcast(a, jnp.bfloat16)
                                 * plsc.bitcast(b, jnp.bfloat16))
                            accbf = p if accbf is None else accbf + p
                        lo, hi = plsc.unpack(
                            accbf, format=plsc.PackFormat.INTERLEAVED)
                        acc0 = acc0 + lo
                        acc1 = acc1 + hi
                    return acc0, acc1

                zero = jnp.zeros((_L,), jnp.float32)
                acc0, acc1 = lax.fori_loop(0, d_words // 16, quad, (zero, zero))
                sco[pl.ds(cs + g * _L, _L)] = acc0 + acc1
                return carry2

            lax.fori_loop(0, n_groups, group_body, 0)

        start_gather(0, sbuf0, dbuf0, sem_s0, sem_d0, chunk)

        def pair_body(c2, carry):
            a = 2 * c2
            start_gather(a + 1, sbuf1, dbuf1, sem_s1, sem_d1, chunk)
            wait_gather(sbuf0, dbuf0, sem_s0, sem_d0, chunk)
            compute(a * chunk, sbuf0, dbuf0, chunk // _L)

            @pl.when(a + 2 < n_full)
            def _prefetch_full():
                start_gather(a + 2, sbuf0, dbuf0, sem_s0, sem_d0, chunk)

            @pl.when(a + 2 == n_full)
            def _prefetch_tail():
                start_gather(n_full, sbuf0, dbuf0, sem_s0, sem_d0, tail)

            wait_gather(sbuf1, dbuf1, sem_s1, sem_d1, chunk)
            compute((a + 1) * chunk, sbuf1, dbuf1, chunk // _L)
            return carry

        lax.fori_loop(0, n_full // 2, pair_body, 0)

        wait_gather(sbuf0, dbuf0, sem_s0, sem_d0, tail)
        compute(n_full * chunk, sbuf0, dbuf0, tail // _L)

        pltpu.sync_copy(sco, out_hbm.at[pl.ds(base, e_per)])

    return sc_kernel(h_packed, src, dst)


# speedup vs baseline: 11.5724x; 1.0097x over previous
import functools

import jax
import jax.numpy as jnp
from jax import lax
from jax.experimental import pallas as pl
from jax.experimental.pallas import tpu as pltpu
from jax.experimental.pallas import tpu_sc as plsc

_NC = 2
_NS = 16
_NW = _NC * _NS
_L = 16


def kernel(h, edge_index):
    n_nodes, d_feat = h.shape
    n_edges = edge_index.shape[1]
    src = edge_index[0].astype(jnp.int32)
    dst = edge_index[1].astype(jnp.int32)
    d_words = d_feat // 2
    h_packed = lax.bitcast_convert_type(
        h.astype(jnp.bfloat16).reshape(n_nodes, d_words, 2), jnp.int32)

    e_per = n_edges // _NW
    chunk = 224
    n_full = e_per // chunk
    tail = e_per - n_full * chunk

    mesh = plsc.VectorSubcoreMesh(core_axis_name="c", subcore_axis_name="s")

    @functools.partial(
        pl.kernel,
        out_type=jax.ShapeDtypeStruct((n_edges,), jnp.float32),
        mesh=mesh,
        scratch_types=[
            pltpu.VMEM((e_per,), jnp.int32),
            pltpu.VMEM((e_per,), jnp.int32),
            pltpu.VMEM((chunk, d_words), jnp.int32),
            pltpu.VMEM((chunk, d_words), jnp.int32),
            pltpu.VMEM((chunk, d_words), jnp.int32),
            pltpu.VMEM((chunk, d_words), jnp.int32),
            pltpu.VMEM((e_per,), jnp.float32),
            pltpu.VMEM_SHARED((n_nodes, d_words), jnp.int32),
            pltpu.SemaphoreType.DMA,
            pltpu.SemaphoreType.DMA,
            pltpu.SemaphoreType.DMA,
            pltpu.SemaphoreType.DMA,
        ],
        compiler_params=pltpu.CompilerParams(
            needs_layout_passes=False, use_tc_tiling_on_sc=False),
    )
    def sc_kernel(h_hbm, src_hbm, dst_hbm, out_hbm,
                  src_v, dst_v, sbuf0, dbuf0, sbuf1, dbuf1, sco, h_spmem,
                  sem_s0, sem_d0, sem_s1, sem_d1):
        sid = lax.axis_index("s")
        wid = sid * _NC + lax.axis_index("c")
        base = wid * e_per
        cp_si = pltpu.async_copy(src_hbm.at[pl.ds(base, e_per)], src_v, sem_s0)
        cp_di = pltpu.async_copy(dst_hbm.at[pl.ds(base, e_per)], dst_v, sem_d0)

        rows_per_sub = n_nodes // _NS
        pltpu.sync_copy(h_hbm.at[pl.ds(sid * rows_per_sub, rows_per_sub)],
                        h_spmem.at[pl.ds(sid * rows_per_sub, rows_per_sub)])
        cp_si.wait()
        cp_di.wait()
        plsc.subcore_barrier()

        lane = lax.iota(jnp.int32, _L)
        lane_xor = [lane ^ j for j in range(8)]

        def start_gather(c, sbuf, dbuf, sem_s, sem_d, size):
            cs = c * chunk
            pltpu.async_copy(h_spmem.at[src_v.at[pl.ds(cs, size)]],
                             sbuf.at[pl.ds(0, size)], sem_s)
            pltpu.async_copy(h_spmem.at[dst_v.at[pl.ds(cs, size)]],
                             dbuf.at[pl.ds(0, size)], sem_d)

        def wait_gather(sbuf, dbuf, sem_s, sem_d, size):
            pltpu.make_async_copy(h_spmem.at[src_v.at[pl.ds(0, size)]],
                                  sbuf.at[pl.ds(0, size)], sem_s).wait()
            pltpu.make_async_copy(h_spmem.at[dst_v.at[pl.ds(0, size)]],
                                  dbuf.at[pl.ds(0, size)], sem_d).wait()

        def compute(cs, srows, drows, n_groups):
            def group_body(g, carry2):
                rows = lane + g * _L

                def quad(o, accs):
                    acc0, acc1 = accs
                    for half in range(2):
                        wbase = o * 16 + half * 8
                        accbf = None
                        for j in range(8):
                            cols = lane_xor[j] ^ wbase
                            a = plsc.load_gather(srows, [rows, cols])
                            b = plsc.load_gather(drows, [rows, cols])
                            p = (plsc.bitcast(a, jnp.bfloat16)
                                 * plsc.bitcast(b, jnp.bfloat16))
                            accbf = p if accbf is None else accbf + p
                        lo, hi = plsc.unpack(
                            accbf, format=plsc.PackFormat.INTERLEAVED)
                        acc0 = acc0 + lo
                        acc1 = acc1 + hi
                    return acc0, acc1

                zero = jnp.zeros((_L,), jnp.float32)
                acc0, acc1 = lax.fori_loop(0, d_words // 16, quad, (zero, zero))
                sco[pl.ds(cs + g * _L, _L)] = acc0 + acc1
                return carry2

            lax.fori_loop(0, n_groups, group_body, 0)

        start_gather(0, sbuf0, dbuf0, sem_s0, sem_d0, chunk)

        def pair_body(c2, carry):
            a = 2 * c2
            start_gather(a + 1, sbuf1, dbuf1, sem_s1, sem_d1, chunk)
            wait_gather(sbuf0, dbuf0, sem_s0, sem_d0, chunk)
            compute(a * chunk, sbuf0, dbuf0, chunk // _L)

            @pl.when(a + 2 < n_full)
            def _prefetch_full():
                start_gather(a + 2, sbuf0, dbuf0, sem_s0, sem_d0, chunk)

            @pl.when(a + 2 == n_full)
            def _prefetch_tail():
                start_gather(n_full, sbuf0, dbuf0, sem_s0, sem_d0, tail)

            wait_gather(sbuf1, dbuf1, sem_s1, sem_d1, chunk)
            compute((a + 1) * chunk, sbuf1, dbuf1, chunk // _L)
            return carry

        lax.fori_loop(0, n_full // 2, pair_body, 0)

        wait_gather(sbuf0, dbuf0, sem_s0, sem_d0, tail)
        compute(n_full * chunk, sbuf0, dbuf0, tail // _L)

        pltpu.sync_copy(sco, out_hbm.at[pl.ds(base, e_per)])

    return sc_kernel(h_packed, src, dst)


# speedup vs baseline: 11.6256x; 1.0046x over previous
import functools

import jax
import jax.numpy as jnp
from jax import lax
from jax.experimental import pallas as pl
from jax.experimental.pallas import tpu as pltpu
from jax.experimental.pallas import tpu_sc as plsc

_NC = 2
_NS = 16
_NW = _NC * _NS
_L = 16


def kernel(h, edge_index):
    n_nodes, d_feat = h.shape
    n_edges = edge_index.shape[1]
    src = edge_index[0].astype(jnp.int32)
    dst = edge_index[1].astype(jnp.int32)
    d_words = d_feat // 2
    h_packed = lax.bitcast_convert_type(
        h.astype(jnp.bfloat16).reshape(n_nodes, d_words, 2), jnp.int32)

    e_per = n_edges // _NW
    chunk = 224
    n_full = e_per // chunk
    tail = e_per - n_full * chunk

    mesh = plsc.VectorSubcoreMesh(core_axis_name="c", subcore_axis_name="s")

    @functools.partial(
        pl.kernel,
        out_type=jax.ShapeDtypeStruct((n_edges,), jnp.float32),
        mesh=mesh,
        scratch_types=[
            pltpu.VMEM((e_per,), jnp.int32),
            pltpu.VMEM((e_per,), jnp.int32),
            pltpu.VMEM((chunk, d_words), jnp.int32),
            pltpu.VMEM((chunk, d_words), jnp.int32),
            pltpu.VMEM((chunk, d_words), jnp.int32),
            pltpu.VMEM((chunk, d_words), jnp.int32),
            pltpu.VMEM((e_per,), jnp.float32),
            pltpu.VMEM_SHARED((n_nodes, d_words), jnp.int32),
            pltpu.SemaphoreType.DMA,
            pltpu.SemaphoreType.DMA,
            pltpu.SemaphoreType.DMA,
            pltpu.SemaphoreType.DMA,
        ],
        compiler_params=pltpu.CompilerParams(
            needs_layout_passes=False, use_tc_tiling_on_sc=False),
    )
    def sc_kernel(h_hbm, src_hbm, dst_hbm, out_hbm,
                  src_v, dst_v, sbuf0, dbuf0, sbuf1, dbuf1, sco, h_spmem,
                  sem_s0, sem_d0, sem_s1, sem_d1):
        sid = lax.axis_index("s")
        wid = sid * _NC + lax.axis_index("c")
        base = wid * e_per
        cp_si = pltpu.async_copy(src_hbm.at[pl.ds(base, e_per)], src_v, sem_s0)
        cp_di = pltpu.async_copy(dst_hbm.at[pl.ds(base, e_per)], dst_v, sem_d0)

        rows_per_sub = n_nodes // _NS
        pltpu.sync_copy(h_hbm.at[pl.ds(sid * rows_per_sub, rows_per_sub)],
                        h_spmem.at[pl.ds(sid * rows_per_sub, rows_per_sub)])
        cp_si.wait()
        cp_di.wait()
        plsc.subcore_barrier()

        lane = lax.iota(jnp.int32, _L)
        lane_xor = [lane ^ j for j in range(16)]

        def start_gather(c, sbuf, dbuf, sem_s, sem_d, size):
            cs = c * chunk
            pltpu.async_copy(h_spmem.at[src_v.at[pl.ds(cs, size)]],
                             sbuf.at[pl.ds(0, size)], sem_s)
            pltpu.async_copy(h_spmem.at[dst_v.at[pl.ds(cs, size)]],
                             dbuf.at[pl.ds(0, size)], sem_d)

        def wait_gather(sbuf, dbuf, sem_s, sem_d, size):
            pltpu.make_async_copy(h_spmem.at[src_v.at[pl.ds(0, size)]],
                                  sbuf.at[pl.ds(0, size)], sem_s).wait()
            pltpu.make_async_copy(h_spmem.at[dst_v.at[pl.ds(0, size)]],
                                  dbuf.at[pl.ds(0, size)], sem_d).wait()

        def compute(cs, srows, drows, n_groups):
            def group_body(g, carry2):
                rows = lane + g * _L

                def quad(o, accs):
                    acc0, acc1 = accs
                    wbase = o * 16
                    acc_a = None
                    acc_b = None
                    for j in range(16):
                        cols = lane_xor[j] ^ wbase
                        a = plsc.load_gather(srows, [rows, cols])
                        b = plsc.load_gather(drows, [rows, cols])
                        p = (plsc.bitcast(a, jnp.bfloat16)
                             * plsc.bitcast(b, jnp.bfloat16))
                        if j % 2 == 0:
                            acc_a = p if acc_a is None else acc_a + p
                        else:
                            acc_b = p if acc_b is None else acc_b + p
                    lo, hi = plsc.unpack(
                        acc_a + acc_b, format=plsc.PackFormat.INTERLEAVED)
                    acc0 = acc0 + lo
                    acc1 = acc1 + hi
                    return acc0, acc1

                zero = jnp.zeros((_L,), jnp.float32)
                acc0, acc1 = lax.fori_loop(0, d_words // 16, quad, (zero, zero))
                sco[pl.ds(cs + g * _L, _L)] = acc0 + acc1
                return carry2

            lax.fori_loop(0, n_groups, group_body, 0)

        start_gather(0, sbuf0, dbuf0, sem_s0, sem_d0, chunk)

        def pair_body(c2, carry):
            a = 2 * c2
            start_gather(a + 1, sbuf1, dbuf1, sem_s1, sem_d1, chunk)
            wait_gather(sbuf0, dbuf0, sem_s0, sem_d0, chunk)
            compute(a * chunk, sbuf0, dbuf0, chunk // _L)

            @pl.when(a + 2 < n_full)
            def _prefetch_full():
                start_gather(a + 2, sbuf0, dbuf0, sem_s0, sem_d0, chunk)

            @pl.when(a + 2 == n_full)
            def _prefetch_tail():
                start_gather(n_full, sbuf0, dbuf0, sem_s0, sem_d0, tail)

            wait_gather(sbuf1, dbuf1, sem_s1, sem_d1, chunk)
            compute((a + 1) * chunk, sbuf1, dbuf1, chunk // _L)
            return carry

        lax.fori_loop(0, n_full // 2, pair_body, 0)

        wait_gather(sbuf0, dbuf0, sem_s0, sem_d0, tail)
        compute(n_full * chunk, sbuf0, dbuf0, tail // _L)

        pltpu.sync_copy(sco, out_hbm.at[pl.ds(base, e_per)])

    return sc_kernel(h_packed, src, dst)
